# initial kernel scaffold (unmeasured)
import jax
import jax.numpy as jnp
from jax import lax
from jax.experimental import pallas as pl
from jax.experimental.pallas import tpu as pltpu

N_DEV = 4


def kernel(x, w_mat):
    m_per, k = x.shape
    _, n_per = w_mat.shape
    half = m_per // 2

    xb = x.astype(jnp.bfloat16).reshape(2, half, k)
    wb = w_mat.astype(jnp.bfloat16)

    def body(x_ref, w_ref, out_ref, gather_ref, send_sems, recv_sems):
        my = lax.axis_index("i")
        left = lax.rem(my + N_DEV - 1, N_DEV)
        right = lax.rem(my + 1, N_DEV)
        opp = lax.rem(my + 2, N_DEV)

        barrier_sem = pltpu.get_barrier_semaphore()
        for nbr in (left, right):
            pl.semaphore_signal(
                barrier_sem, inc=1,
                device_id=(nbr,), device_id_type=pl.DeviceIdType.MESH,
            )
        pl.semaphore_wait(barrier_sem, 2)

        rdma_a = pltpu.make_async_remote_copy(
            src_ref=x_ref, dst_ref=gather_ref.at[2],
            send_sem=send_sems.at[0], recv_sem=recv_sems.at[0],
            device_id=(right,), device_id_type=pl.DeviceIdType.MESH,
        )
        rdma_b = pltpu.make_async_remote_copy(
            src_ref=x_ref, dst_ref=gather_ref.at[0],
            send_sem=send_sems.at[1], recv_sem=recv_sems.at[1],
            device_id=(left,), device_id_type=pl.DeviceIdType.MESH,
        )
        rdma_a.start()
        rdma_b.start()

        def mm(src):
            return jnp.maximum(
                jnp.dot(src, w_ref[:, :], preferred_element_type=jnp.float32),
                0.0,
            )

        def store(origin, h, val):
            out_ref[pl.ds(origin * m_per + h * half, half), :] = val

        store(my, 0, mm(x_ref[0]))
        store(my, 1, mm(x_ref[1]))

        rdma_a.wait_recv()
        rdma_c = pltpu.make_async_remote_copy(
            src_ref=gather_ref.at[2, 0], dst_ref=gather_ref.at[1, 0],
            send_sem=send_sems.at[2], recv_sem=recv_sems.at[2],
            device_id=(right,), device_id_type=pl.DeviceIdType.MESH,
        )
        rdma_c.start()
        rdma_b.wait_recv()
        rdma_d = pltpu.make_async_remote_copy(
            src_ref=gather_ref.at[0, 1], dst_ref=gather_ref.at[1, 1],
            send_sem=send_sems.at[3], recv_sem=recv_sems.at[3],
            device_id=(left,), device_id_type=pl.DeviceIdType.MESH,
        )
        rdma_d.start()

        store(left, 0, mm(gather_ref[2, 0]))
        store(left, 1, mm(gather_ref[2, 1]))
        store(right, 0, mm(gather_ref[0, 0]))
        store(right, 1, mm(gather_ref[0, 1]))

        rdma_c.wait_recv()
        store(opp, 0, mm(gather_ref[1, 0]))
        rdma_d.wait_recv()
        store(opp, 1, mm(gather_ref[1, 1]))

        rdma_a.wait_send()
        rdma_b.wait_send()
        rdma_c.wait_send()
        rdma_d.wait_send()

    return pl.pallas_call(
        body,
        out_shape=jax.ShapeDtypeStruct((N_DEV * m_per, n_per), jnp.float32),
        in_specs=[
            pl.BlockSpec(memory_space=pltpu.VMEM),
            pl.BlockSpec(memory_space=pltpu.VMEM),
        ],
        out_specs=pl.BlockSpec(memory_space=pltpu.VMEM),
        scratch_shapes=[
            pltpu.VMEM((3, 2, half, k), jnp.bfloat16),
            pltpu.SemaphoreType.DMA((4,)),
            pltpu.SemaphoreType.DMA((4,)),
        ],
        compiler_params=pltpu.CompilerParams(collective_id=0),
    )(xb, wb)


# baseline (device time: 220369 ns/iter reference)
import jax
import jax.numpy as jnp
from jax import lax
from jax.experimental import pallas as pl
from jax.experimental.pallas import tpu as pltpu

N_DEV = 4


def kernel(x, w_mat):
    m_per, k = x.shape
    _, n_per = w_mat.shape
    half = m_per // 2

    xb = x.astype(jnp.bfloat16).reshape(2, half, k)
    wb = w_mat.astype(jnp.bfloat16)

    def body(x_ref, w_ref, out_ref, gather_ref, ostage_ref,
             send_sems, recv_sems, copy_sems):
        my = lax.axis_index("i")
        left = lax.rem(my + N_DEV - 1, N_DEV)
        right = lax.rem(my + 1, N_DEV)
        opp = lax.rem(my + 2, N_DEV)

        barrier_sem = pltpu.get_barrier_semaphore()
        for nbr in (left, right):
            pl.semaphore_signal(
                barrier_sem, inc=1,
                device_id=(nbr,), device_id_type=pl.DeviceIdType.MESH,
            )
        pl.semaphore_wait(barrier_sem, 2)

        rdma_a = pltpu.make_async_remote_copy(
            src_ref=x_ref, dst_ref=gather_ref.at[2],
            send_sem=send_sems.at[0], recv_sem=recv_sems.at[0],
            device_id=(right,), device_id_type=pl.DeviceIdType.MESH,
        )
        rdma_b = pltpu.make_async_remote_copy(
            src_ref=x_ref, dst_ref=gather_ref.at[0],
            send_sem=send_sems.at[1], recv_sem=recv_sems.at[1],
            device_id=(left,), device_id_type=pl.DeviceIdType.MESH,
        )
        rdma_a.start()
        rdma_b.start()

        def mm(src):
            return jnp.maximum(
                jnp.dot(src, w_ref[:, :], preferred_element_type=jnp.float32),
                0.0,
            )

        pending = [None, None]
        counter = [0]

        def store(origin, h, val):
            slot = counter[0] % 2
            counter[0] += 1
            if pending[slot] is not None:
                pending[slot].wait()
            ostage_ref[slot] = val
            cp = pltpu.make_async_copy(
                ostage_ref.at[slot],
                out_ref.at[pl.ds(origin * m_per + h * half, half), :],
                copy_sems.at[slot],
            )
            cp.start()
            pending[slot] = cp

        store(my, 0, mm(x_ref[0]))
        store(my, 1, mm(x_ref[1]))

        rdma_a.wait_recv()
        rdma_c = pltpu.make_async_remote_copy(
            src_ref=gather_ref.at[2, 0], dst_ref=gather_ref.at[1, 0],
            send_sem=send_sems.at[2], recv_sem=recv_sems.at[2],
            device_id=(right,), device_id_type=pl.DeviceIdType.MESH,
        )
        rdma_c.start()
        rdma_b.wait_recv()
        rdma_d = pltpu.make_async_remote_copy(
            src_ref=gather_ref.at[0, 1], dst_ref=gather_ref.at[1, 1],
            send_sem=send_sems.at[3], recv_sem=recv_sems.at[3],
            device_id=(left,), device_id_type=pl.DeviceIdType.MESH,
        )
        rdma_d.start()

        store(left, 0, mm(gather_ref[2, 0]))
        store(left, 1, mm(gather_ref[2, 1]))
        store(right, 0, mm(gather_ref[0, 0]))
        store(right, 1, mm(gather_ref[0, 1]))

        rdma_c.wait_recv()
        store(opp, 0, mm(gather_ref[1, 0]))
        rdma_d.wait_recv()
        store(opp, 1, mm(gather_ref[1, 1]))

        for cp in pending:
            if cp is not None:
                cp.wait()
        rdma_a.wait_send()
        rdma_b.wait_send()
        rdma_c.wait_send()
        rdma_d.wait_send()

    return pl.pallas_call(
        body,
        out_shape=jax.ShapeDtypeStruct((N_DEV * m_per, n_per), jnp.float32),
        in_specs=[
            pl.BlockSpec(memory_space=pltpu.VMEM),
            pl.BlockSpec(memory_space=pltpu.VMEM),
        ],
        out_specs=pl.BlockSpec(memory_space=pl.ANY),
        scratch_shapes=[
            pltpu.VMEM((3, 2, half, k), jnp.bfloat16),
            pltpu.VMEM((2, half, n_per), jnp.float32),
            pltpu.SemaphoreType.DMA((4,)),
            pltpu.SemaphoreType.DMA((4,)),
            pltpu.SemaphoreType.DMA((2,)),
        ],
        compiler_params=pltpu.CompilerParams(
            collective_id=0,
            vmem_limit_bytes=67_000_000,
        ),
    )(xb, wb)


# device time: 210535 ns/iter; 1.0467x vs baseline; 1.0467x over previous
import jax
import jax.numpy as jnp
from jax import lax
from jax.experimental import pallas as pl
from jax.experimental.pallas import tpu as pltpu

N_DEV = 4
W_TILES = 8
W_CAST_ROWS = 1024


def kernel(x, w_mat):
    m_per, k = x.shape
    _, n_per = w_mat.shape
    half = m_per // 2
    n_tile = n_per // W_TILES

    xb = x.astype(jnp.bfloat16).reshape(2, half, k)
    wb = w_mat.astype(jnp.bfloat16)

    def body(x_ref, wb_ref, out_ref, gather_ref,
             ostage_ref, send_sems, recv_sems, copy_sems):
        my = lax.axis_index("i")
        left = lax.rem(my + N_DEV - 1, N_DEV)
        right = lax.rem(my + 1, N_DEV)
        opp = lax.rem(my + 2, N_DEV)

        barrier_sem = pltpu.get_barrier_semaphore()
        for nbr in (left, right):
            pl.semaphore_signal(
                barrier_sem, inc=1,
                device_id=(nbr,), device_id_type=pl.DeviceIdType.MESH,
            )
        pl.semaphore_wait(barrier_sem, 2)

        rdma_a = pltpu.make_async_remote_copy(
            src_ref=x_ref, dst_ref=gather_ref.at[2],
            send_sem=send_sems.at[0], recv_sem=recv_sems.at[0],
            device_id=(right,), device_id_type=pl.DeviceIdType.MESH,
        )
        rdma_b = pltpu.make_async_remote_copy(
            src_ref=x_ref, dst_ref=gather_ref.at[0],
            send_sem=send_sems.at[1], recv_sem=recv_sems.at[1],
            device_id=(left,), device_id_type=pl.DeviceIdType.MESH,
        )
        rdma_a.start()
        rdma_b.start()

        def mm(src):
            acc = jnp.dot(src, wb_ref[:, :], preferred_element_type=jnp.float32)
            return jnp.maximum(acc, 0.0).astype(jnp.bfloat16)

        pending = [None, None]
        counter = [0]

        def store(origin, h, val):
            slot = counter[0] % 2
            counter[0] += 1
            if pending[slot] is not None:
                pending[slot].wait()
            ostage_ref[slot] = val
            cp = pltpu.make_async_copy(
                ostage_ref.at[slot],
                out_ref.at[pl.ds(origin * m_per + h * half, half), :],
                copy_sems.at[slot],
            )
            cp.start()
            pending[slot] = cp

        store(my, 0, mm(x_ref[0]))
        store(my, 1, mm(x_ref[1]))

        rdma_a.wait_recv()
        rdma_c = pltpu.make_async_remote_copy(
            src_ref=gather_ref.at[2, 0], dst_ref=gather_ref.at[1, 0],
            send_sem=send_sems.at[2], recv_sem=recv_sems.at[2],
            device_id=(right,), device_id_type=pl.DeviceIdType.MESH,
        )
        rdma_c.start()
        rdma_b.wait_recv()
        rdma_d = pltpu.make_async_remote_copy(
            src_ref=gather_ref.at[0, 1], dst_ref=gather_ref.at[1, 1],
            send_sem=send_sems.at[3], recv_sem=recv_sems.at[3],
            device_id=(left,), device_id_type=pl.DeviceIdType.MESH,
        )
        rdma_d.start()

        store(left, 0, mm(gather_ref[2, 0]))
        store(left, 1, mm(gather_ref[2, 1]))
        store(right, 0, mm(gather_ref[0, 0]))
        store(right, 1, mm(gather_ref[0, 1]))

        rdma_c.wait_recv()
        store(opp, 0, mm(gather_ref[1, 0]))
        rdma_d.wait_recv()
        store(opp, 1, mm(gather_ref[1, 1]))

        for cp in pending:
            if cp is not None:
                cp.wait()
        rdma_a.wait_send()
        rdma_b.wait_send()
        rdma_c.wait_send()
        rdma_d.wait_send()

    return pl.pallas_call(
        body,
        out_shape=jax.ShapeDtypeStruct((N_DEV * m_per, n_per), jnp.bfloat16),
        in_specs=[
            pl.BlockSpec(memory_space=pltpu.VMEM),
            pl.BlockSpec(memory_space=pltpu.VMEM),
        ],
        out_specs=pl.BlockSpec(memory_space=pl.ANY),
        scratch_shapes=[
            pltpu.VMEM((3, 2, half, k), jnp.bfloat16),
            pltpu.VMEM((2, half, n_per), jnp.bfloat16),
            pltpu.SemaphoreType.DMA((4,)),
            pltpu.SemaphoreType.DMA((4,)),
            pltpu.SemaphoreType.DMA((2,)),
        ],
        compiler_params=pltpu.CompilerParams(
            collective_id=0,
            vmem_limit_bytes=67_000_000,
        ),
    )(xb, wb)


# device time: 195625 ns/iter; 1.1265x vs baseline; 1.0762x over previous
import os

import jax

_CACHE_DIR = os.path.join(os.path.dirname(os.path.abspath(__file__)), ".jax_cache")
jax.config.update("jax_compilation_cache_dir", _CACHE_DIR)
jax.config.update("jax_persistent_cache_min_compile_time_secs", 0)
jax.config.update("jax_persistent_cache_min_entry_size_bytes", 0)

import jax.numpy as jnp
from jax import lax
from jax.experimental import pallas as pl
from jax.experimental.pallas import tpu as pltpu

N_DEV = 4
N_Q = 8


def kernel(x, w_mat):
    m_per, k = x.shape
    _, n_per = w_mat.shape
    q = m_per // N_Q

    xb = x.astype(jnp.bfloat16)
    wb = w_mat.astype(jnp.bfloat16)

    def body(x_ref, wb_ref, out_ref, gather_ref,
             ostage_ref, send_sems, recv_sems, copy_sems):
        my = lax.axis_index("i")
        left = lax.rem(my + N_DEV - 1, N_DEV)
        right = lax.rem(my + 1, N_DEV)
        opp = lax.rem(my + 2, N_DEV)

        barrier_sem = pltpu.get_barrier_semaphore()
        for nbr in (left, right):
            pl.semaphore_signal(
                barrier_sem, inc=1,
                device_id=(nbr,), device_id_type=pl.DeviceIdType.MESH,
            )
        pl.semaphore_wait(barrier_sem, 2)

        def rdma(src, dst_slot, rows, dev, sem):
            return pltpu.make_async_remote_copy(
                src_ref=src,
                dst_ref=gather_ref.at[dst_slot, pl.ds(rows, q)],
                send_sem=send_sems.at[sem], recv_sem=recv_sems.at[sem],
                device_id=(dev,), device_id_type=pl.DeviceIdType.MESH,
            )

        a_rdmas = []
        b_rdmas = []
        for i in range(N_Q):
            a = rdma(x_ref.at[pl.ds(i * q, q)], 2, i * q, right, i)
            b = rdma(x_ref.at[pl.ds(i * q, q)], 0, i * q, left, N_Q + i)
            a.start()
            b.start()
            a_rdmas.append(a)
            b_rdmas.append(b)

        def mm(src):
            acc = jnp.dot(src, wb_ref[:, :], preferred_element_type=jnp.float32)
            return jnp.maximum(acc, 0.0).astype(jnp.bfloat16)

        pending = [None, None]
        counter = [0]

        def store(origin, i, val):
            slot = counter[0] % 2
            counter[0] += 1
            if pending[slot] is not None:
                pending[slot].wait()
            ostage_ref[slot] = val
            cp = pltpu.make_async_copy(
                ostage_ref.at[slot],
                out_ref.at[pl.ds(origin * m_per + i * q, q), :],
                copy_sems.at[slot],
            )
            cp.start()
            pending[slot] = cp

        for i in range(N_Q):
            store(my, i, mm(x_ref[pl.ds(i * q, q)]))

        c_rdmas = []
        d_rdmas = []
        for i in range(N_Q):
            a_rdmas[i].wait_recv()
            if i < N_Q // 2:
                c = rdma(gather_ref.at[2, pl.ds(i * q, q)], 1, i * q,
                         right, 2 * N_Q + i)
                c.start()
                c_rdmas.append(c)
            store(left, i, mm(gather_ref[2, pl.ds(i * q, q)]))

            b_rdmas[i].wait_recv()
            if i >= N_Q // 2:
                d = rdma(gather_ref.at[0, pl.ds(i * q, q)], 1, i * q,
                         left, 2 * N_Q + i)
                d.start()
                d_rdmas.append(d)
            store(right, i, mm(gather_ref[0, pl.ds(i * q, q)]))

        for j in range(N_Q // 2):
            c_rdmas[j].wait_recv()
            store(opp, j, mm(gather_ref[1, pl.ds(j * q, q)]))
            d_rdmas[j].wait_recv()
            jj = N_Q // 2 + j
            store(opp, jj, mm(gather_ref[1, pl.ds(jj * q, q)]))

        for cp in pending:
            if cp is not None:
                cp.wait()
        for r in a_rdmas + b_rdmas + c_rdmas + d_rdmas:
            r.wait_send()

    return pl.pallas_call(
        body,
        out_shape=jax.ShapeDtypeStruct((N_DEV * m_per, n_per), jnp.bfloat16),
        in_specs=[
            pl.BlockSpec(memory_space=pltpu.VMEM),
            pl.BlockSpec(memory_space=pltpu.VMEM),
        ],
        out_specs=pl.BlockSpec(memory_space=pl.ANY),
        scratch_shapes=[
            pltpu.VMEM((3, m_per, k), jnp.bfloat16),
            pltpu.VMEM((2, q, n_per), jnp.bfloat16),
            pltpu.SemaphoreType.DMA((3 * N_Q,)),
            pltpu.SemaphoreType.DMA((3 * N_Q,)),
            pltpu.SemaphoreType.DMA((2,)),
        ],
        compiler_params=pltpu.CompilerParams(
            collective_id=0,
            vmem_limit_bytes=67_000_000,
        ),
    )(xb, wb)
